# baseline (speedup 1.0000x reference)
import jax
import jax.numpy as jnp
from jax import lax
from jax.experimental import pallas as pl
from jax.experimental.pallas import tpu as pltpu

N = 4096
D = 128
M = 128
T = 32
MET = 14
BR = 256
NB = N // BR
NLAYERS = 4


def _gcn_steps(s, adj_ref, x_ref, w_ref, b_ref, adj_bf, feat, support):
    @pl.when(s == 0)
    def _support0():
        support[...] = jnp.dot(
            x_ref[...], w_ref[0].astype(jnp.bfloat16),
            preferred_element_type=jnp.float32).astype(jnp.bfloat16)

    @pl.when(s < NB)
    def _layer0():
        ablk = adj_ref[...].astype(jnp.bfloat16)
        adj_bf[pl.ds(s * BR, BR), :] = ablk
        acc = jnp.dot(ablk, support[...], preferred_element_type=jnp.float32)
        feat[pl.ds(s * BR, BR), :] = jnp.maximum(acc + b_ref[0], 0.0)

    @pl.when(s >= NB)
    def _layers123():
        support[...] = jnp.dot(
            feat[...].astype(jnp.bfloat16), w_ref[0].astype(jnp.bfloat16),
            preferred_element_type=jnp.float32).astype(jnp.bfloat16)
        acc = jnp.dot(adj_bf[...], support[...],
                      preferred_element_type=jnp.float32)
        feat[...] = jnp.maximum(acc + b_ref[0], 0.0)


def _attention_embed(feat, wa_ref):
    emb = feat[...]
    ridx = lax.broadcasted_iota(jnp.int32, (N, 1), 0)
    valid = ridx < (N - 1)
    colsum = jnp.sum(jnp.where(valid, emb, 0.0), axis=0, keepdims=True)
    gc = jnp.dot(colsum, wa_ref[...],
                 preferred_element_type=jnp.float32) * (1.0 / (N - 1))
    tg = jnp.tanh(gc)
    logits = jnp.sum(emb * tg, axis=1, keepdims=True)
    sig = jnp.where(valid, jax.nn.sigmoid(logits), 0.0)
    return jnp.sum(emb * sig, axis=0, keepdims=True)


def _stack_body_b(adj_ref, x_ref, w_ref, b_ref, wa_ref, e_ref,
                  adj_bf, feat, support):
    s = pl.program_id(0)
    _gcn_steps(s, adj_ref, x_ref, w_ref, b_ref, adj_bf, feat, support)

    @pl.when(s == pl.num_programs(0) - 1)
    def _attention():
        e_ref[...] = _attention_embed(feat, wa_ref)


def _readout_body(e1_ref, e2_ref, wtc_ref, summat_ref, wtb1_ref, wtb2_ref,
                  tb_ref, met_ref, fcwa_ref, fcwm_ref, fcb_ref,
                  out_ref, agg_ref):
    e1 = e1_ref[...]
    e2 = e2_ref[...]
    p2r = jnp.dot(e2, wtc_ref[...], preferred_element_type=jnp.float32)
    e1til = jnp.concatenate([e1] * T, axis=1)
    v = p2r * e1til
    s_row = jnp.dot(v, summat_ref[...], preferred_element_type=jnp.float32)
    block = (jnp.dot(e1, wtb1_ref[...], preferred_element_type=jnp.float32)
             + jnp.dot(e2, wtb2_ref[...], preferred_element_type=jnp.float32))
    scores = jnp.maximum(s_row + block + tb_ref[...], 0.0)
    agg_ref[...] = scores
    out_ref[...] = (
        jnp.sum(scores * fcwa_ref[...], axis=1, keepdims=True)
        + jnp.sum(met_ref[...] * fcwm_ref[...], axis=1, keepdims=True)
        + fcb_ref[...])


_GCN_SPECS = [
    pl.BlockSpec((BR, N), lambda s: (jnp.where(s < NB, s, NB - 1), 0)),
    pl.BlockSpec((N, D), lambda s: (0, 0)),
    pl.BlockSpec((1, D, M), lambda s: (jnp.where(s < NB, 0, s - NB + 1), 0, 0)),
    pl.BlockSpec((1, 1, M), lambda s: (jnp.where(s < NB, 0, s - NB + 1), 0, 0)),
    pl.BlockSpec((M, M), lambda s: (0, 0)),
]

_GCN_SCRATCH = [
    pltpu.VMEM((N, N), jnp.bfloat16),
    pltpu.VMEM((N, M), jnp.float32),
    pltpu.VMEM((N, M), jnp.bfloat16),
]


def _full(shape):
    return pl.BlockSpec(shape, lambda s: tuple(0 for _ in shape))


def kernel(b_x, b_adj, a_x, a_adj, metrics,
           W11, b11, W12, b12, W13, b13, W14, b14,
           W21, b21, W22, b22, W23, b23, W24, b24,
           Wa, Wt, Wtb, tbias, fcW, fcb):
    wstack_b = jnp.stack((W11, W12, W13, W14))
    bstack_b = jnp.stack((b11, b12, b13, b14)).reshape(NLAYERS, 1, M)
    wstack_a = jnp.stack((W21, W22, W23, W24))
    bstack_a = jnp.stack((b21, b22, b23, b24)).reshape(NLAYERS, 1, M)

    e1 = pl.pallas_call(
        _stack_body_b,
        grid=(NB + NLAYERS - 1,),
        in_specs=_GCN_SPECS,
        out_specs=_full((1, M)),
        out_shape=jax.ShapeDtypeStruct((1, M), jnp.float32),
        scratch_shapes=_GCN_SCRATCH,
        compiler_params=pltpu.CompilerParams(
            dimension_semantics=("arbitrary",)),
    )(b_adj, b_x.astype(jnp.bfloat16), wstack_b, bstack_b, Wa)

    e2 = pl.pallas_call(
        _stack_body_b,
        grid=(NB + NLAYERS - 1,),
        in_specs=_GCN_SPECS,
        out_specs=_full((1, M)),
        out_shape=jax.ShapeDtypeStruct((1, M), jnp.float32),
        scratch_shapes=_GCN_SCRATCH,
        compiler_params=pltpu.CompilerParams(
            dimension_semantics=("arbitrary",)),
    )(a_adj, a_x.astype(jnp.bfloat16), wstack_a, bstack_a, Wa)

    wt_cols = jnp.transpose(Wt, (1, 2, 0)).reshape(M, T * M)
    summat = jnp.repeat(jnp.eye(T, dtype=jnp.float32), M, axis=0)
    wtb1 = Wtb[:, :M].T
    wtb2 = Wtb[:, M:].T

    out2, agg2 = pl.pallas_call(
        _readout_body,
        out_shape=[
            jax.ShapeDtypeStruct((1, 1), jnp.float32),
            jax.ShapeDtypeStruct((1, T), jnp.float32),
        ],
    )(e1, e2, wt_cols, summat, wtb1, wtb2, tbias.reshape(1, T),
      metrics.reshape(1, MET), fcW[:, :T], fcW[:, T:], fcb.reshape(1, 1))
    return (out2.reshape(1), agg2.reshape(T))


# speedup vs baseline: 1.0032x; 1.0032x over previous
import jax
import jax.numpy as jnp
from jax import lax
from jax.experimental import pallas as pl
from jax.experimental.pallas import tpu as pltpu

N = 4096
D = 128
M = 128
T = 32
MET = 14
BR = 256
NB = N // BR
NLAYERS = 4


def _gcn_steps(s, adj_ref, x_ref, w_ref, b_ref, adj_bf, feat, support):
    @pl.when(s == 0)
    def _support0():
        support[...] = jnp.dot(
            x_ref[...], w_ref[0].astype(jnp.bfloat16),
            preferred_element_type=jnp.float32).astype(jnp.bfloat16)

    @pl.when(s < NB)
    def _layer0():
        ablk = adj_ref[...].astype(jnp.bfloat16)
        adj_bf[pl.ds(s * BR, BR), :] = ablk
        acc = jnp.dot(ablk, support[...], preferred_element_type=jnp.float32)
        feat[pl.ds(s * BR, BR), :] = jnp.maximum(acc + b_ref[0], 0.0)

    @pl.when(s >= NB)
    def _layers123():
        support[...] = jnp.dot(
            feat[...].astype(jnp.bfloat16), w_ref[0].astype(jnp.bfloat16),
            preferred_element_type=jnp.float32).astype(jnp.bfloat16)
        acc = jnp.dot(adj_bf[...], support[...],
                      preferred_element_type=jnp.float32)
        feat[...] = jnp.maximum(acc + b_ref[0], 0.0)


def _attention_embed(feat, wa_ref):
    emb = feat[...]
    ridx = lax.broadcasted_iota(jnp.int32, (N, 1), 0)
    valid = ridx < (N - 1)
    colsum = jnp.sum(jnp.where(valid, emb, 0.0), axis=0, keepdims=True)
    gc = jnp.dot(colsum, wa_ref[...],
                 preferred_element_type=jnp.float32) * (1.0 / (N - 1))
    tg = jnp.tanh(gc)
    logits = jnp.sum(emb * tg, axis=1, keepdims=True)
    sig = jnp.where(valid, jax.nn.sigmoid(logits), 0.0)
    return jnp.sum(emb * sig, axis=0, keepdims=True)


def _stack_body_b(adj_ref, x_ref, w_ref, b_ref, wa_ref, feat, e_ref,
                  adj_bf, support):
    s = pl.program_id(0)
    _gcn_steps(s, adj_ref, x_ref, w_ref, b_ref, adj_bf, feat, support)

    @pl.when(s == pl.num_programs(0) - 1)
    def _attention():
        e_ref[...] = _attention_embed(feat, wa_ref)


def _readout_body(e1_ref, e2_ref, wtc_ref, summat_ref, wtb1_ref, wtb2_ref,
                  tb_ref, met_ref, fcwa_ref, fcwm_ref, fcb_ref,
                  out_ref, agg_ref):
    e1 = e1_ref[...]
    e2 = e2_ref[...]
    p2r = jnp.dot(e2, wtc_ref[...], preferred_element_type=jnp.float32)
    e1til = jnp.concatenate([e1] * T, axis=1)
    v = p2r * e1til
    s_row = jnp.dot(v, summat_ref[...], preferred_element_type=jnp.float32)
    block = (jnp.dot(e1, wtb1_ref[...], preferred_element_type=jnp.float32)
             + jnp.dot(e2, wtb2_ref[...], preferred_element_type=jnp.float32))
    scores = jnp.maximum(s_row + block + tb_ref[...], 0.0)
    agg_ref[...] = scores
    out_ref[...] = (
        jnp.sum(scores * fcwa_ref[...], axis=1, keepdims=True)
        + jnp.sum(met_ref[...] * fcwm_ref[...], axis=1, keepdims=True)
        + fcb_ref[...])


_GCN_SPECS = [
    pl.BlockSpec((BR, N), lambda s: (jnp.where(s < NB, s, NB - 1), 0)),
    pl.BlockSpec((N, D), lambda s: (0, 0)),
    pl.BlockSpec((1, D, M), lambda s: (jnp.where(s < NB, 0, s - NB + 1), 0, 0)),
    pl.BlockSpec((1, 1, M), lambda s: (jnp.where(s < NB, 0, s - NB + 1), 0, 0)),
    pl.BlockSpec((M, M), lambda s: (0, 0)),
]

_GCN_SCRATCH = [
    pltpu.VMEM((N, N), jnp.bfloat16),
    pltpu.VMEM((N, M), jnp.bfloat16),
]


def _full(shape):
    return pl.BlockSpec(shape, lambda s: tuple(0 for _ in shape))


def kernel(b_x, b_adj, a_x, a_adj, metrics,
           W11, b11, W12, b12, W13, b13, W14, b14,
           W21, b21, W22, b22, W23, b23, W24, b24,
           Wa, Wt, Wtb, tbias, fcW, fcb):
    wstack_b = jnp.stack((W11, W12, W13, W14))
    bstack_b = jnp.stack((b11, b12, b13, b14)).reshape(NLAYERS, 1, M)
    wstack_a = jnp.stack((W21, W22, W23, W24))
    bstack_a = jnp.stack((b21, b22, b23, b24)).reshape(NLAYERS, 1, M)

    e1 = pl.pallas_call(
        _stack_body_b,
        grid=(NB + NLAYERS - 1,),
        in_specs=_GCN_SPECS,
        out_specs=[_full((N, M)), _full((1, M))],
        out_shape=[jax.ShapeDtypeStruct((N, M), jnp.float32),
                   jax.ShapeDtypeStruct((1, M), jnp.float32)],
        scratch_shapes=_GCN_SCRATCH,
        compiler_params=pltpu.CompilerParams(
            dimension_semantics=("arbitrary",)),
    )(b_adj, b_x.astype(jnp.bfloat16), wstack_b, bstack_b, Wa)[1]

    e2 = pl.pallas_call(
        _stack_body_b,
        grid=(NB + NLAYERS - 1,),
        in_specs=_GCN_SPECS,
        out_specs=[_full((N, M)), _full((1, M))],
        out_shape=[jax.ShapeDtypeStruct((N, M), jnp.float32),
                   jax.ShapeDtypeStruct((1, M), jnp.float32)],
        scratch_shapes=_GCN_SCRATCH,
        compiler_params=pltpu.CompilerParams(
            dimension_semantics=("arbitrary",)),
    )(a_adj, a_x.astype(jnp.bfloat16), wstack_a, bstack_a, Wa)[1]

    wt_cols = jnp.transpose(Wt, (1, 2, 0)).reshape(M, T * M)
    summat = jnp.repeat(jnp.eye(T, dtype=jnp.float32), M, axis=0)
    wtb1 = Wtb[:, :M].T
    wtb2 = Wtb[:, M:].T

    out2, agg2 = pl.pallas_call(
        _readout_body,
        out_shape=[
            jax.ShapeDtypeStruct((1, 1), jnp.float32),
            jax.ShapeDtypeStruct((1, T), jnp.float32),
        ],
    )(e1, e2, wt_cols, summat, wtb1, wtb2, tbias.reshape(1, T),
      metrics.reshape(1, MET), fcW[:, :T], fcW[:, T:], fcb.reshape(1, 1))
    return (out2.reshape(1), agg2.reshape(T))


# speedup vs baseline: 1.0316x; 1.0283x over previous
import jax
import jax.numpy as jnp
from jax import lax
from jax.experimental import pallas as pl
from jax.experimental.pallas import tpu as pltpu

N = 4096
D = 128
M = 128
T = 32
MET = 14
BR = 256
NB = N // BR
NLAYERS = 4


def _gcn_steps(s, adj_ref, x_ref, w_ref, b_ref, adj_bf, feat, support):
    @pl.when(s == 0)
    def _support0():
        support[...] = jnp.dot(
            x_ref[...], w_ref[0].astype(jnp.bfloat16),
            preferred_element_type=jnp.float32).astype(jnp.bfloat16)

    @pl.when(s < NB)
    def _layer0():
        ablk = adj_ref[...].astype(jnp.bfloat16)
        adj_bf[pl.ds(s * BR, BR), :] = ablk
        acc = jnp.dot(ablk, support[...], preferred_element_type=jnp.float32)
        feat[pl.ds(s * BR, BR), :] = jnp.maximum(acc + b_ref[0], 0.0)

    @pl.when(s >= NB)
    def _layers123():
        support[...] = jnp.dot(
            feat[...].astype(jnp.bfloat16), w_ref[0].astype(jnp.bfloat16),
            preferred_element_type=jnp.float32).astype(jnp.bfloat16)
        acc = jnp.dot(adj_bf[...], support[...],
                      preferred_element_type=jnp.float32)
        feat[...] = jnp.maximum(acc + b_ref[0], 0.0)


def _attention_embed(feat, wa_ref):
    emb = feat[...]
    ridx = lax.broadcasted_iota(jnp.int32, (N, 1), 0)
    valid = ridx < (N - 1)
    colsum = jnp.sum(jnp.where(valid, emb, 0.0), axis=0, keepdims=True)
    gc = jnp.dot(colsum, wa_ref[...],
                 preferred_element_type=jnp.float32) * (1.0 / (N - 1))
    tg = jnp.tanh(gc)
    logits = jnp.sum(emb * tg, axis=1, keepdims=True)
    sig = jnp.where(valid, jax.nn.sigmoid(logits), 0.0)
    return jnp.sum(emb * sig, axis=0, keepdims=True)


def _stack_body_b(adj_ref, x_ref, w_ref, b_ref, wa_ref, feat, e_ref,
                  adj_bf, support):
    s = pl.program_id(0)
    _gcn_steps(s, adj_ref, x_ref, w_ref, b_ref, adj_bf, feat, support)

    @pl.when(s == pl.num_programs(0) - 1)
    def _attention():
        e_ref[...] = _attention_embed(feat, wa_ref)


def _readout_body(e1_ref, e2_ref, wt_ref, wtb1_ref, wtb2_ref,
                  tb_ref, met_ref, fcwa_ref, fcwm_ref, fcb_ref,
                  out_ref, agg_ref):
    e1 = e1_ref[...]
    e2 = e2_ref[...]
    outer = lax.dot_general(e1, e2, (((0,), (0,)), ((), ())),
                            preferred_element_type=jnp.float32)
    lane = lax.broadcasted_iota(jnp.int32, (1, T), 1)
    s_row = jnp.zeros((1, T), jnp.float32)
    for t in range(T):
        s_t = jnp.sum(wt_ref[t] * outer)
        s_row = s_row + jnp.where(lane == t, s_t, 0.0)
    block = (jnp.dot(e1, wtb1_ref[...], preferred_element_type=jnp.float32)
             + jnp.dot(e2, wtb2_ref[...], preferred_element_type=jnp.float32))
    scores = jnp.maximum(s_row + block + tb_ref[...], 0.0)
    agg_ref[...] = scores
    out_ref[...] = (
        jnp.sum(scores * fcwa_ref[...], axis=1, keepdims=True)
        + jnp.sum(met_ref[...] * fcwm_ref[...], axis=1, keepdims=True)
        + fcb_ref[...])


_GCN_SPECS = [
    pl.BlockSpec((BR, N), lambda s: (jnp.where(s < NB, s, NB - 1), 0)),
    pl.BlockSpec((N, D), lambda s: (0, 0)),
    pl.BlockSpec((1, D, M), lambda s: (jnp.where(s < NB, 0, s - NB + 1), 0, 0)),
    pl.BlockSpec((1, 1, M), lambda s: (jnp.where(s < NB, 0, s - NB + 1), 0, 0)),
    pl.BlockSpec((M, M), lambda s: (0, 0)),
]

_GCN_SCRATCH = [
    pltpu.VMEM((N, N), jnp.bfloat16),
    pltpu.VMEM((N, M), jnp.bfloat16),
]


def _full(shape):
    return pl.BlockSpec(shape, lambda s: tuple(0 for _ in shape))


def kernel(b_x, b_adj, a_x, a_adj, metrics,
           W11, b11, W12, b12, W13, b13, W14, b14,
           W21, b21, W22, b22, W23, b23, W24, b24,
           Wa, Wt, Wtb, tbias, fcW, fcb):
    wstack_b = jnp.stack((W11, W12, W13, W14))
    bstack_b = jnp.stack((b11, b12, b13, b14)).reshape(NLAYERS, 1, M)
    wstack_a = jnp.stack((W21, W22, W23, W24))
    bstack_a = jnp.stack((b21, b22, b23, b24)).reshape(NLAYERS, 1, M)

    e1 = pl.pallas_call(
        _stack_body_b,
        grid=(NB + NLAYERS - 1,),
        in_specs=_GCN_SPECS,
        out_specs=[_full((N, M)), _full((1, M))],
        out_shape=[jax.ShapeDtypeStruct((N, M), jnp.float32),
                   jax.ShapeDtypeStruct((1, M), jnp.float32)],
        scratch_shapes=_GCN_SCRATCH,
        compiler_params=pltpu.CompilerParams(
            dimension_semantics=("arbitrary",)),
    )(b_adj, b_x.astype(jnp.bfloat16), wstack_b, bstack_b, Wa)[1]

    e2 = pl.pallas_call(
        _stack_body_b,
        grid=(NB + NLAYERS - 1,),
        in_specs=_GCN_SPECS,
        out_specs=[_full((N, M)), _full((1, M))],
        out_shape=[jax.ShapeDtypeStruct((N, M), jnp.float32),
                   jax.ShapeDtypeStruct((1, M), jnp.float32)],
        scratch_shapes=_GCN_SCRATCH,
        compiler_params=pltpu.CompilerParams(
            dimension_semantics=("arbitrary",)),
    )(a_adj, a_x.astype(jnp.bfloat16), wstack_a, bstack_a, Wa)[1]

    wt_p = jnp.transpose(Wt, (2, 0, 1))
    wtb1 = Wtb[:, :M].T
    wtb2 = Wtb[:, M:].T

    out2, agg2 = pl.pallas_call(
        _readout_body,
        out_shape=[
            jax.ShapeDtypeStruct((1, 1), jnp.float32),
            jax.ShapeDtypeStruct((1, T), jnp.float32),
        ],
    )(e1, e2, wt_p, wtb1, wtb2, tbias.reshape(1, T),
      metrics.reshape(1, MET), fcW[:, :T], fcW[:, T:], fcb.reshape(1, 1))
    return (out2.reshape(1), agg2.reshape(T))
